# baseline (speedup 1.0000x reference)
import functools

import jax
import jax.numpy as jnp
from jax import lax
from jax.experimental import pallas as pl
from jax.experimental.pallas import tpu as pltpu
from jax.experimental.pallas import tpu_sc as plsc

_EMBED_DIM = 768
_NGRIDS = 256
_LANES = 16

_NC = 2
_NS = 16
_NW = _NC * _NS

_CHUNK = 64


def _slide_pe_body(n_tokens, x_hbm, c0_hbm, c1_hbm, table_hbm, out_hbm,
                   c0_v, c1_v, idx_v, xb, gb, sem):
    rows_per_w = n_tokens // _NW
    n_chunks = rows_per_w // _CHUNK
    wid = lax.axis_index("s") * _NC + lax.axis_index("c")
    base = wid * rows_per_w

    pltpu.sync_copy(c0_hbm.at[pl.ds(base, rows_per_w)], c0_v)
    pltpu.sync_copy(c1_hbm.at[pl.ds(base, rows_per_w)], c1_v)

    def idx_body(i, _):
        ci = i // (_CHUNK // _LANES)
        off = (i % (_CHUNK // _LANES)) * _LANES
        a = c0_v[pl.ds(i * _LANES, _LANES)]
        b = c1_v[pl.ds(i * _LANES, _LANES)]
        ga = ((a >> 5) * 9363) >> 16
        gb16 = ((b >> 5) * 9363) >> 16
        idx_v[ci, pl.ds(off, _LANES)] = ga * _NGRIDS + gb16
        return 0

    lax.fori_loop(0, rows_per_w // _LANES, idx_body, 0)

    def chunk_body(ci, _):
        row0 = base + ci * _CHUNK
        gather = pltpu.async_copy(table_hbm.at[idx_v.at[ci]], gb, sem)
        pltpu.sync_copy(x_hbm.at[pl.ds(row0, _CHUNK)], xb)
        gather.wait()

        def add_body(r, _):
            for j in range(_EMBED_DIM // _LANES):
                sl = pl.ds(j * _LANES, _LANES)
                xb[r, sl] = xb[r, sl] + gb[r, sl]
            return 0

        lax.fori_loop(0, _CHUNK, add_body, 0)
        pltpu.sync_copy(xb, out_hbm.at[pl.ds(row0, _CHUNK)])
        return 0

    lax.fori_loop(0, n_chunks, chunk_body, 0)


@jax.jit
def kernel(x, coords, pos_embed):
    b, n, d = x.shape
    n_tokens = b * n
    x2d = x.reshape(n_tokens, d)
    ci32 = coords.astype(jnp.int32)
    c0 = ci32[..., 0].reshape(n_tokens)
    c1 = ci32[..., 1].reshape(n_tokens)
    table = pos_embed[0]

    mesh = plsc.VectorSubcoreMesh(core_axis_name="c", subcore_axis_name="s")
    rows_per_w = n_tokens // _NW
    run = pl.kernel(
        functools.partial(_slide_pe_body, n_tokens),
        out_type=jax.ShapeDtypeStruct((n_tokens, d), jnp.float32),
        mesh=mesh,
        scratch_types=[
            pltpu.VMEM((rows_per_w,), jnp.int32),
            pltpu.VMEM((rows_per_w,), jnp.int32),
            pltpu.VMEM((rows_per_w // _CHUNK, _CHUNK), jnp.int32),
            pltpu.VMEM((_CHUNK, _EMBED_DIM), jnp.float32),
            pltpu.VMEM((_CHUNK, _EMBED_DIM), jnp.float32),
            pltpu.SemaphoreType.DMA,
        ],
    )
    out = run(x2d, c0, c1, table)
    return out.reshape(b, n, d)
